# baseline (speedup 1.0000x reference)
import functools

import jax
import jax.numpy as jnp
from jax import lax
from jax.experimental import pallas as pl
from jax.experimental.pallas import tpu as pltpu
from jax.experimental.pallas import tpu_sc as plsc

CUTOFF = 5.0
AD = 16
FD = 128
N_NODES = 10000
N_EDGES = 320000
N_ANGLES = 2000000

SUB = 128
BROWS = 4
BLK = BROWS * SUB
N_BLK = N_ANGLES // BLK
TAIL_ROW = N_ANGLES // SUB - 1
RSZ = 80000
TRASH = RSZ
WB = 5000

_MESH = plsc.VectorSubcoreMesh(core_axis_name="c", subcore_axis_name="s",
                               num_cores=2, num_subcores=16)


def _filter_body(a_ref, w_ref, b_ref, o_ref):
    o_ref[...] = jax.nn.sigmoid(
        jnp.dot(a_ref[...], w_ref[...], preferred_element_type=jnp.float32)
        + b_ref[...])


def _tc_filter(atomic_features, W_atom, b_atom2):
    return pl.pallas_call(
        _filter_body,
        grid=(10,),
        in_specs=[
            pl.BlockSpec((1000, FD), lambda i: (i, 0)),
            pl.BlockSpec((FD, AD), lambda i: (0, 0)),
            pl.BlockSpec((1, AD), lambda i: (0, 0)),
        ],
        out_specs=pl.BlockSpec((1000, AD), lambda i: (i, 0)),
        out_shape=jax.ShapeDtypeStruct((N_NODES, AD), jnp.float32),
    )(atomic_features, W_atom, b_atom2)


def _env_body(d_ref, o_ref):
    r = d_ref[...] * (1.0 / CUTOFF)
    r3 = r * r * r
    env = 1.0 - r3 * ((6.0 * r - 15.0) * r + 10.0)
    o_ref[...] = jnp.broadcast_to(env, (4000, AD))


def _tc_envrows(d_col):
    return pl.pallas_call(
        _env_body,
        grid=(80,),
        in_specs=[pl.BlockSpec((4000, 1), lambda i: (i, 0))],
        out_specs=pl.BlockSpec((4000, AD), lambda i: (i, 0)),
        out_shape=jax.ShapeDtypeStruct((N_EDGES, AD), jnp.float32),
    )(d_col)


def _split_body(t_ref, ij_ref, ik_ref):
    x = t_ref[...].astype(jnp.float32)
    r = lax.broadcasted_iota(jnp.int32, (2 * SUB, SUB), 0)
    c = lax.broadcasted_iota(jnp.int32, (2 * SUB, SUB), 1)
    s_even = (r == 2 * c).astype(jnp.float32)
    s_odd = (r == 2 * c + 1).astype(jnp.float32)
    ij_ref[...] = jnp.dot(x, s_even,
                          preferred_element_type=jnp.float32).astype(jnp.int32)
    ik_ref[...] = jnp.dot(x, s_odd,
                          preferred_element_type=jnp.float32).astype(jnp.int32)


def _tc_split(tbi2):
    return pl.pallas_call(
        _split_body,
        grid=(16,),
        in_specs=[pl.BlockSpec((1000, 2 * SUB), lambda i: (i, 0))],
        out_specs=[pl.BlockSpec((1000, SUB), lambda i: (i, 0)),
                   pl.BlockSpec((1000, SUB), lambda i: (i, 0))],
        out_shape=[jax.ShapeDtypeStruct((16000, SUB), jnp.int32),
                   jax.ShapeDtypeStruct((16000, SUB), jnp.int32)],
    )(tbi2)


def _final_body(acc_ref, env_ref, ef_ref, wg_ref, bg_ref, wt_ref, bt_ref,
                o_ref):
    t = acc_ref[...] * env_ref[...]
    h = jnp.dot(t, wg_ref[...], preferred_element_type=jnp.float32) + bg_ref[...]
    swish = h * jax.nn.sigmoid(h)
    gate = jax.nn.sigmoid(
        jnp.dot(t, wt_ref[...], preferred_element_type=jnp.float32) + bt_ref[...])
    o_ref[...] = ef_ref[...] + swish * gate


def _tc_final(acc, env_col, edge_features, W_g, b_g2, W_gate, b_gate2):
    return pl.pallas_call(
        _final_body,
        grid=(625,),
        in_specs=[
            pl.BlockSpec((512, AD), lambda i: (i, 0)),
            pl.BlockSpec((512, AD), lambda i: (i, 0)),
            pl.BlockSpec((512, FD), lambda i: (i, 0)),
            pl.BlockSpec((AD, FD), lambda i: (0, 0)),
            pl.BlockSpec((1, FD), lambda i: (0, 0)),
            pl.BlockSpec((AD, FD), lambda i: (0, 0)),
            pl.BlockSpec((1, FD), lambda i: (0, 0)),
        ],
        out_specs=pl.BlockSpec((512, FD), lambda i: (i, 0)),
        out_shape=jax.ShapeDtypeStruct((N_EDGES, FD), jnp.float32),
    )(acc, env_col, edge_features, W_g, b_g2, W_gate, b_gate2)


@functools.partial(
    pl.kernel,
    out_type=jax.ShapeDtypeStruct((N_EDGES, AD), jnp.float32),
    mesh=_MESH,
    scratch_types=[
        pltpu.VMEM((SUB,), jnp.int32),
        pltpu.VMEM((SUB, AD), jnp.float32),
        pltpu.VMEM((SUB, AD), jnp.float32),
        pltpu.VMEM((SUB, AD), jnp.float32),
        pltpu.SemaphoreType.DMA,
    ],
    compiler_params=pltpu.CompilerParams(use_tc_tiling_on_sc=False),
)
def _sc_g(filt_hbm, envr_hbm, kn_hbm, g_hbm, kn_v, envr_v, afk_v, g_v, sem):
    c = lax.axis_index("c")
    s = lax.axis_index("s")
    w = c * 16 + s
    n_chunks = N_EDGES // SUB

    def chunk(i, carry):
        cidx = w + 32 * i

        @pl.when(cidx < n_chunks)
        def _():
            base = cidx * SUB
            pltpu.sync_copy(kn_hbm.at[pl.ds(base, SUB)], kn_v)
            pltpu.sync_copy(envr_hbm.at[pl.ds(base, SUB)], envr_v)
            pltpu.async_copy(filt_hbm.at[kn_v], afk_v, sem).wait()

            def row(e, carry2):
                e8 = e * 8
                for k in range(8):
                    g_v[e8 + k, :] = afk_v[e8 + k, :] * envr_v[e8 + k, :]
                return carry2

            lax.fori_loop(0, SUB // 8, row, 0)
            pltpu.sync_copy(g_v, g_hbm.at[pl.ds(base, SUB)])

        return carry

    lax.fori_loop(0, (n_chunks + 31) // 32, chunk, 0)


@functools.partial(
    pl.kernel,
    out_type=jax.ShapeDtypeStruct((N_EDGES, AD), jnp.float32),
    mesh=_MESH,
    scratch_types=[
        pltpu.VMEM((BROWS, SUB), jnp.int32),
        pltpu.VMEM((BROWS, SUB), jnp.int32),
        pltpu.VMEM((BLK, AD), jnp.float32),
        pltpu.VMEM((BLK, AD), jnp.float32),
        pltpu.VMEM((500, AD), jnp.float32),
        pltpu.VMEM_SHARED((RSZ + 8, AD), jnp.float32),
        pltpu.SemaphoreType.DMA,
        pltpu.SemaphoreType.DMA,
    ],
    compiler_params=pltpu.CompilerParams(use_tc_tiling_on_sc=False),
)
def _sc_scatter(g_hbm, af_hbm, ij_hbm, ik_hbm, acc_hbm,
                ij_v, ik_v, af_v, g_v, zero_v, acc_sp, sem, gsem):
    c = lax.axis_index("c")
    s = lax.axis_index("s")
    iota16 = lax.broadcasted_iota(jnp.int32, (16,), 0)
    trash16 = TRASH + (iota16 & 7)
    zv = jnp.zeros((16,), jnp.float32)

    def zrow(r, carry):
        zero_v[r, :] = zv
        return carry

    lax.fori_loop(0, 500, zrow, 0)

    def do_block(row0, nrows, lo):
        pltpu.sync_copy(ij_hbm.at[pl.ds(row0, nrows)], ij_v.at[pl.ds(0, nrows)])
        pltpu.sync_copy(ik_hbm.at[pl.ds(row0, nrows)], ik_v.at[pl.ds(0, nrows)])
        af_src = af_hbm.at[pl.ds(row0 * SUB, nrows * SUB)]
        af_dst = af_v.at[pl.ds(0, nrows * SUB)]
        pltpu.async_copy(af_src, af_dst, sem)
        for j in range(nrows):
            pltpu.async_copy(g_hbm.at[ik_v.at[j]],
                             g_v.at[pl.ds(j * SUB, SUB)], gsem)
        for v in range(nrows * (SUB // 16)):
            j, sl = v // (SUB // 16), (v % (SUB // 16)) * 16
            rel = ij_v[j, pl.ds(sl, 16)] - lo
            msk = (rel >= 0) & (rel < RSZ)
            ij_v[j, pl.ds(sl, 16)] = jnp.where(msk, rel, trash16)
        for j in range(nrows):
            pltpu.make_async_copy(g_hbm.at[ik_v.at[j]],
                                  g_v.at[pl.ds(j * SUB, SUB)], gsem).wait()
        pltpu.make_async_copy(af_src, af_dst, sem).wait()

        def prow(r, carry):
            r8 = r * 8
            for k in range(8):
                g_v[r8 + k, :] = af_v[r8 + k, :] * g_v[r8 + k, :]
            return carry

        lax.fori_loop(0, nrows * SUB // 8, prow, 0)
        for j in range(nrows):
            pltpu.sync_copy(g_v.at[pl.ds(j * SUB, SUB)],
                            acc_sp.at[ij_v.at[j]], add=True)

    for scan in range(2):
        rng = 2 * scan + c
        lo = rng * RSZ
        for q in range(WB // 500):
            pltpu.sync_copy(zero_v, acc_sp.at[pl.ds(s * WB + q * 500, 500)])

        @pl.when(s == 0)
        def _():
            pltpu.sync_copy(zero_v.at[pl.ds(0, 8)], acc_sp.at[pl.ds(RSZ, 8)])

        plsc.subcore_barrier()

        def blk_loop(i, carry):
            b = s + 16 * i

            @pl.when(b < N_BLK)
            def _():
                do_block(b * BROWS, BROWS, lo)

            return carry

        lax.fori_loop(0, (N_BLK + 15) // 16, blk_loop, 0)

        @pl.when(s == 15)
        def _():
            do_block(TAIL_ROW, 1, lo)

        plsc.subcore_barrier()
        for q in range(WB // 500):
            off = s * WB + q * 500
            pltpu.sync_copy(acc_sp.at[pl.ds(off, 500)],
                            acc_hbm.at[pl.ds(lo + off, 500)])
        plsc.subcore_barrier()


def kernel(atomic_features, edge_features, angle_features, edge_dist,
           W_atom, b_atom, W_g, b_g, W_gate, b_gate,
           edge_index, three_body_indices_with_offset):
    filt = _tc_filter(atomic_features, W_atom, b_atom.reshape(1, AD))
    envr = _tc_envrows(edge_dist.reshape(N_EDGES, 1))
    kn = edge_index[1].astype(jnp.int32)
    g = _sc_g(filt, envr, kn)
    tbi2 = three_body_indices_with_offset.astype(jnp.int32).reshape(
        N_ANGLES // SUB, 2 * SUB)
    tbi2 = jnp.pad(tbi2, ((0, 16000 - N_ANGLES // SUB), (0, 0)))
    ij, ik = _tc_split(tbi2)
    acc = _sc_scatter(g, angle_features, ij[:N_ANGLES // SUB],
                      ik[:N_ANGLES // SUB])
    return _tc_final(acc, envr, edge_features,
                     W_g, b_g.reshape(1, FD), W_gate, b_gate.reshape(1, FD))


# speedup vs baseline: 1.0218x; 1.0218x over previous
import functools

import jax
import jax.numpy as jnp
from jax import lax
from jax.experimental import pallas as pl
from jax.experimental.pallas import tpu as pltpu
from jax.experimental.pallas import tpu_sc as plsc

CUTOFF = 5.0
AD = 16
FD = 128
N_NODES = 10000
N_EDGES = 320000
N_ANGLES = 2000000

SUB = 128
BROWS = 4
BLK = BROWS * SUB
N_BLK = N_ANGLES // BLK
TAIL_ROW = N_ANGLES // SUB - 1
RSZ = 80000
TRASH = RSZ
WB = 5000

_MESH = plsc.VectorSubcoreMesh(core_axis_name="c", subcore_axis_name="s",
                               num_cores=2, num_subcores=16)


def _filter_body(a_ref, w_ref, b_ref, o_ref):
    o_ref[...] = jax.nn.sigmoid(
        jnp.dot(a_ref[...], w_ref[...], preferred_element_type=jnp.float32)
        + b_ref[...])


def _tc_filter(atomic_features, W_atom, b_atom2):
    return pl.pallas_call(
        _filter_body,
        grid=(10,),
        in_specs=[
            pl.BlockSpec((1000, FD), lambda i: (i, 0)),
            pl.BlockSpec((FD, AD), lambda i: (0, 0)),
            pl.BlockSpec((1, AD), lambda i: (0, 0)),
        ],
        out_specs=pl.BlockSpec((1000, AD), lambda i: (i, 0)),
        out_shape=jax.ShapeDtypeStruct((N_NODES, AD), jnp.float32),
    )(atomic_features, W_atom, b_atom2)


def _env_body(d_ref, o_ref):
    r = d_ref[...] * (1.0 / CUTOFF)
    r3 = r * r * r
    env = 1.0 - r3 * ((6.0 * r - 15.0) * r + 10.0)
    o_ref[...] = jnp.broadcast_to(env, (4000, AD))


def _tc_envrows(d_col):
    return pl.pallas_call(
        _env_body,
        grid=(80,),
        in_specs=[pl.BlockSpec((4000, 1), lambda i: (i, 0))],
        out_specs=pl.BlockSpec((4000, AD), lambda i: (i, 0)),
        out_shape=jax.ShapeDtypeStruct((N_EDGES, AD), jnp.float32),
    )(d_col)


def _split_body(t_ref, ij_ref, ik_ref):
    x = t_ref[...].astype(jnp.float32)
    r = lax.broadcasted_iota(jnp.int32, (2 * SUB, SUB), 0)
    c = lax.broadcasted_iota(jnp.int32, (2 * SUB, SUB), 1)
    s_even = (r == 2 * c).astype(jnp.float32)
    s_odd = (r == 2 * c + 1).astype(jnp.float32)
    ij_ref[...] = jnp.dot(x, s_even, preferred_element_type=jnp.float32,
                          precision=lax.Precision.HIGHEST).astype(jnp.int32)
    ik_ref[...] = jnp.dot(x, s_odd, preferred_element_type=jnp.float32,
                          precision=lax.Precision.HIGHEST).astype(jnp.int32)


def _tc_split(tbi2):
    return pl.pallas_call(
        _split_body,
        grid=(16,),
        in_specs=[pl.BlockSpec((1000, 2 * SUB), lambda i: (i, 0))],
        out_specs=[pl.BlockSpec((1000, SUB), lambda i: (i, 0)),
                   pl.BlockSpec((1000, SUB), lambda i: (i, 0))],
        out_shape=[jax.ShapeDtypeStruct((16000, SUB), jnp.int32),
                   jax.ShapeDtypeStruct((16000, SUB), jnp.int32)],
    )(tbi2)


def _final_body(acc_ref, env_ref, ef_ref, wg_ref, bg_ref, wt_ref, bt_ref,
                o_ref):
    t = acc_ref[...] * env_ref[...]
    h = jnp.dot(t, wg_ref[...], preferred_element_type=jnp.float32) + bg_ref[...]
    swish = h * jax.nn.sigmoid(h)
    gate = jax.nn.sigmoid(
        jnp.dot(t, wt_ref[...], preferred_element_type=jnp.float32) + bt_ref[...])
    o_ref[...] = ef_ref[...] + swish * gate


def _tc_final(acc, env_col, edge_features, W_g, b_g2, W_gate, b_gate2):
    return pl.pallas_call(
        _final_body,
        grid=(625,),
        in_specs=[
            pl.BlockSpec((512, AD), lambda i: (i, 0)),
            pl.BlockSpec((512, AD), lambda i: (i, 0)),
            pl.BlockSpec((512, FD), lambda i: (i, 0)),
            pl.BlockSpec((AD, FD), lambda i: (0, 0)),
            pl.BlockSpec((1, FD), lambda i: (0, 0)),
            pl.BlockSpec((AD, FD), lambda i: (0, 0)),
            pl.BlockSpec((1, FD), lambda i: (0, 0)),
        ],
        out_specs=pl.BlockSpec((512, FD), lambda i: (i, 0)),
        out_shape=jax.ShapeDtypeStruct((N_EDGES, FD), jnp.float32),
    )(acc, env_col, edge_features, W_g, b_g2, W_gate, b_gate2)


@functools.partial(
    pl.kernel,
    out_type=jax.ShapeDtypeStruct((N_EDGES, AD), jnp.float32),
    mesh=_MESH,
    scratch_types=[
        pltpu.VMEM((SUB,), jnp.int32),
        pltpu.VMEM((SUB, AD), jnp.float32),
        pltpu.VMEM((SUB, AD), jnp.float32),
        pltpu.VMEM((SUB, AD), jnp.float32),
        pltpu.SemaphoreType.DMA,
    ],
    compiler_params=pltpu.CompilerParams(use_tc_tiling_on_sc=False),
)
def _sc_g(filt_hbm, envr_hbm, kn_hbm, g_hbm, kn_v, envr_v, afk_v, g_v, sem):
    c = lax.axis_index("c")
    s = lax.axis_index("s")
    w = c * 16 + s
    n_chunks = N_EDGES // SUB

    def chunk(i, carry):
        cidx = w + 32 * i

        @pl.when(cidx < n_chunks)
        def _():
            base = cidx * SUB
            pltpu.sync_copy(kn_hbm.at[pl.ds(base, SUB)], kn_v)
            pltpu.sync_copy(envr_hbm.at[pl.ds(base, SUB)], envr_v)
            pltpu.async_copy(filt_hbm.at[kn_v], afk_v, sem).wait()

            def row(e, carry2):
                e8 = e * 8
                for k in range(8):
                    g_v[e8 + k, :] = afk_v[e8 + k, :] * envr_v[e8 + k, :]
                return carry2

            lax.fori_loop(0, SUB // 8, row, 0)
            pltpu.sync_copy(g_v, g_hbm.at[pl.ds(base, SUB)])

        return carry

    lax.fori_loop(0, (n_chunks + 31) // 32, chunk, 0)


@functools.partial(
    pl.kernel,
    out_type=jax.ShapeDtypeStruct((N_EDGES, AD), jnp.float32),
    mesh=_MESH,
    scratch_types=[
        pltpu.VMEM((BROWS, SUB), jnp.int32),
        pltpu.VMEM((BROWS, SUB), jnp.int32),
        pltpu.VMEM((BLK, AD), jnp.float32),
        pltpu.VMEM((BLK, AD), jnp.float32),
        pltpu.VMEM((500, AD), jnp.float32),
        pltpu.VMEM_SHARED((RSZ + 8, AD), jnp.float32),
        pltpu.SemaphoreType.DMA,
        pltpu.SemaphoreType.DMA,
    ],
    compiler_params=pltpu.CompilerParams(use_tc_tiling_on_sc=False),
)
def _sc_scatter(g_hbm, af_hbm, ij_hbm, ik_hbm, acc_hbm,
                ij_v, ik_v, af_v, g_v, zero_v, acc_sp, sem, gsem):
    c = lax.axis_index("c")
    s = lax.axis_index("s")
    iota16 = lax.broadcasted_iota(jnp.int32, (16,), 0)
    trash16 = TRASH + (iota16 & 7)
    zv = jnp.zeros((16,), jnp.float32)

    def zrow(r, carry):
        zero_v[r, :] = zv
        return carry

    lax.fori_loop(0, 500, zrow, 0)

    def do_block(row0, nrows, lo):
        pltpu.sync_copy(ij_hbm.at[pl.ds(row0, nrows)], ij_v.at[pl.ds(0, nrows)])
        pltpu.sync_copy(ik_hbm.at[pl.ds(row0, nrows)], ik_v.at[pl.ds(0, nrows)])
        af_src = af_hbm.at[pl.ds(row0 * SUB, nrows * SUB)]
        af_dst = af_v.at[pl.ds(0, nrows * SUB)]
        pltpu.async_copy(af_src, af_dst, sem)
        for j in range(nrows):
            pltpu.async_copy(g_hbm.at[ik_v.at[j]],
                             g_v.at[pl.ds(j * SUB, SUB)], gsem)
        for v in range(nrows * (SUB // 16)):
            j, sl = v // (SUB // 16), (v % (SUB // 16)) * 16
            rel = ij_v[j, pl.ds(sl, 16)] - lo
            msk = (rel >= 0) & (rel < RSZ)
            ij_v[j, pl.ds(sl, 16)] = jnp.where(msk, rel, trash16)
        for j in range(nrows):
            pltpu.make_async_copy(g_hbm.at[ik_v.at[j]],
                                  g_v.at[pl.ds(j * SUB, SUB)], gsem).wait()
        pltpu.make_async_copy(af_src, af_dst, sem).wait()

        def prow(r, carry):
            r8 = r * 8
            for k in range(8):
                g_v[r8 + k, :] = af_v[r8 + k, :] * g_v[r8 + k, :]
            return carry

        lax.fori_loop(0, nrows * SUB // 8, prow, 0)
        for j in range(nrows):
            pltpu.sync_copy(g_v.at[pl.ds(j * SUB, SUB)],
                            acc_sp.at[ij_v.at[j]], add=True)

    for scan in range(2):
        rng = 2 * scan + c
        lo = rng * RSZ
        for q in range(WB // 500):
            pltpu.sync_copy(zero_v, acc_sp.at[pl.ds(s * WB + q * 500, 500)])

        @pl.when(s == 0)
        def _():
            pltpu.sync_copy(zero_v.at[pl.ds(0, 8)], acc_sp.at[pl.ds(RSZ, 8)])

        plsc.subcore_barrier()

        def blk_loop(i, carry):
            b = s + 16 * i

            @pl.when(b < N_BLK)
            def _():
                do_block(b * BROWS, BROWS, lo)

            return carry

        lax.fori_loop(0, (N_BLK + 15) // 16, blk_loop, 0)

        @pl.when(s == 15)
        def _():
            do_block(TAIL_ROW, 1, lo)

        plsc.subcore_barrier()
        for q in range(WB // 500):
            off = s * WB + q * 500
            pltpu.sync_copy(acc_sp.at[pl.ds(off, 500)],
                            acc_hbm.at[pl.ds(lo + off, 500)])
        plsc.subcore_barrier()


def kernel(atomic_features, edge_features, angle_features, edge_dist,
           W_atom, b_atom, W_g, b_g, W_gate, b_gate,
           edge_index, three_body_indices_with_offset):
    filt = _tc_filter(atomic_features, W_atom, b_atom.reshape(1, AD))
    envr = _tc_envrows(edge_dist.reshape(N_EDGES, 1))
    kn = edge_index[1].astype(jnp.int32)
    g = _sc_g(filt, envr, kn)
    tbi2 = three_body_indices_with_offset.astype(jnp.int32).reshape(
        N_ANGLES // SUB, 2 * SUB)
    ij, ik = _tc_split(tbi2)
    acc = _sc_scatter(g, angle_features, ij, ik)
    return _tc_final(acc, envr, edge_features,
                     W_g, b_g.reshape(1, FD), W_gate, b_gate.reshape(1, FD))


# speedup vs baseline: 1.6930x; 1.6568x over previous
import functools

import jax
import jax.numpy as jnp
from jax import lax
from jax.experimental import pallas as pl
from jax.experimental.pallas import tpu as pltpu
from jax.experimental.pallas import tpu_sc as plsc

CUTOFF = 5.0
AD = 16
FD = 128
N_NODES = 10000
N_EDGES = 320000
N_ANGLES = 2000000

SUB = 128
BROWS = 4
BLK = BROWS * SUB
N_BLK = N_ANGLES // BLK
TAIL_ROW = N_ANGLES // SUB - 1
RSZ = 80000
TRASH = RSZ
WB = 5000

_MESH = plsc.VectorSubcoreMesh(core_axis_name="c", subcore_axis_name="s",
                               num_cores=2, num_subcores=16)


def _filter_body(a_ref, w_ref, b_ref, o_ref):
    o_ref[...] = jax.nn.sigmoid(
        jnp.dot(a_ref[...], w_ref[...], preferred_element_type=jnp.float32)
        + b_ref[...])


def _tc_filter(atomic_features, W_atom, b_atom2):
    return pl.pallas_call(
        _filter_body,
        grid=(10,),
        in_specs=[
            pl.BlockSpec((1000, FD), lambda i: (i, 0)),
            pl.BlockSpec((FD, AD), lambda i: (0, 0)),
            pl.BlockSpec((1, AD), lambda i: (0, 0)),
        ],
        out_specs=pl.BlockSpec((1000, AD), lambda i: (i, 0)),
        out_shape=jax.ShapeDtypeStruct((N_NODES, AD), jnp.float32),
    )(atomic_features, W_atom, b_atom2)


def _env_body(d_ref, o_ref):
    r = d_ref[...] * (1.0 / CUTOFF)
    r3 = r * r * r
    env = 1.0 - r3 * ((6.0 * r - 15.0) * r + 10.0)
    o_ref[...] = jnp.broadcast_to(env, (4000, AD))


def _tc_envrows(d_col):
    return pl.pallas_call(
        _env_body,
        grid=(80,),
        in_specs=[pl.BlockSpec((4000, 1), lambda i: (i, 0))],
        out_specs=pl.BlockSpec((4000, AD), lambda i: (i, 0)),
        out_shape=jax.ShapeDtypeStruct((N_EDGES, AD), jnp.float32),
    )(d_col)


def _final_body(acc_ref, env_ref, ef_ref, wg_ref, bg_ref, wt_ref, bt_ref,
                o_ref):
    t = acc_ref[...] * env_ref[...]
    h = jnp.dot(t, wg_ref[...], preferred_element_type=jnp.float32) + bg_ref[...]
    swish = h * jax.nn.sigmoid(h)
    gate = jax.nn.sigmoid(
        jnp.dot(t, wt_ref[...], preferred_element_type=jnp.float32) + bt_ref[...])
    o_ref[...] = ef_ref[...] + swish * gate


def _tc_final(acc, env_col, edge_features, W_g, b_g2, W_gate, b_gate2):
    return pl.pallas_call(
        _final_body,
        grid=(625,),
        in_specs=[
            pl.BlockSpec((512, AD), lambda i: (i, 0)),
            pl.BlockSpec((512, AD), lambda i: (i, 0)),
            pl.BlockSpec((512, FD), lambda i: (i, 0)),
            pl.BlockSpec((AD, FD), lambda i: (0, 0)),
            pl.BlockSpec((1, FD), lambda i: (0, 0)),
            pl.BlockSpec((AD, FD), lambda i: (0, 0)),
            pl.BlockSpec((1, FD), lambda i: (0, 0)),
        ],
        out_specs=pl.BlockSpec((512, FD), lambda i: (i, 0)),
        out_shape=jax.ShapeDtypeStruct((N_EDGES, FD), jnp.float32),
    )(acc, env_col, edge_features, W_g, b_g2, W_gate, b_gate2)


@functools.partial(
    pl.kernel,
    out_type=jax.ShapeDtypeStruct((N_EDGES, AD), jnp.float32),
    mesh=_MESH,
    scratch_types=[
        pltpu.VMEM((SUB,), jnp.int32),
        pltpu.VMEM((SUB, AD), jnp.float32),
        pltpu.VMEM((SUB, AD), jnp.float32),
        pltpu.VMEM((SUB, AD), jnp.float32),
        pltpu.SemaphoreType.DMA,
    ],
    compiler_params=pltpu.CompilerParams(use_tc_tiling_on_sc=False),
)
def _sc_g(filt_hbm, envr_hbm, kn_hbm, g_hbm, kn_v, envr_v, afk_v, g_v, sem):
    c = lax.axis_index("c")
    s = lax.axis_index("s")
    w = c * 16 + s
    n_chunks = N_EDGES // SUB

    def chunk(i, carry):
        cidx = w + 32 * i

        @pl.when(cidx < n_chunks)
        def _():
            base = cidx * SUB
            pltpu.sync_copy(kn_hbm.at[pl.ds(base, SUB)], kn_v)
            pltpu.sync_copy(envr_hbm.at[pl.ds(base, SUB)], envr_v)
            pltpu.async_copy(filt_hbm.at[kn_v], afk_v, sem).wait()

            def row(e, carry2):
                e8 = e * 8
                for k in range(8):
                    g_v[e8 + k, :] = afk_v[e8 + k, :] * envr_v[e8 + k, :]
                return carry2

            lax.fori_loop(0, SUB // 8, row, 0)
            pltpu.sync_copy(g_v, g_hbm.at[pl.ds(base, SUB)])

        return carry

    lax.fori_loop(0, (n_chunks + 31) // 32, chunk, 0)


@functools.partial(
    pl.kernel,
    out_type=jax.ShapeDtypeStruct((N_EDGES, AD), jnp.float32),
    mesh=_MESH,
    scratch_types=[
        pltpu.VMEM((BROWS, SUB), jnp.int32),
        pltpu.VMEM((BROWS, SUB), jnp.int32),
        pltpu.VMEM((BLK, AD), jnp.float32),
        pltpu.VMEM((BLK, AD), jnp.float32),
        pltpu.VMEM((500, AD), jnp.float32),
        pltpu.VMEM_SHARED((RSZ + 8, AD), jnp.float32),
        pltpu.SemaphoreType.DMA,
        pltpu.SemaphoreType.DMA,
    ],
    compiler_params=pltpu.CompilerParams(use_tc_tiling_on_sc=False),
)
def _sc_scatter(g_hbm, af_hbm, ij_hbm, ik_hbm, acc_hbm,
                ij_v, ik_v, af_v, g_v, zero_v, acc_sp, sem, gsem):
    c = lax.axis_index("c")
    s = lax.axis_index("s")
    iota16 = lax.broadcasted_iota(jnp.int32, (16,), 0)
    trash16 = TRASH + (iota16 & 7)
    zv = jnp.zeros((16,), jnp.float32)

    def zrow(r, carry):
        zero_v[r, :] = zv
        return carry

    lax.fori_loop(0, 500, zrow, 0)

    def do_block(row0, nrows, lo):
        pltpu.sync_copy(ij_hbm.at[pl.ds(row0, nrows)], ij_v.at[pl.ds(0, nrows)])
        pltpu.sync_copy(ik_hbm.at[pl.ds(row0, nrows)], ik_v.at[pl.ds(0, nrows)])
        af_src = af_hbm.at[pl.ds(row0 * SUB, nrows * SUB)]
        af_dst = af_v.at[pl.ds(0, nrows * SUB)]
        pltpu.async_copy(af_src, af_dst, sem)
        for j in range(nrows):
            pltpu.async_copy(g_hbm.at[ik_v.at[j]],
                             g_v.at[pl.ds(j * SUB, SUB)], gsem)
        for v in range(nrows * (SUB // 16)):
            j, sl = v // (SUB // 16), (v % (SUB // 16)) * 16
            rel = ij_v[j, pl.ds(sl, 16)] - lo
            msk = (rel >= 0) & (rel < RSZ)
            ij_v[j, pl.ds(sl, 16)] = jnp.where(msk, rel, trash16)
        for j in range(nrows):
            pltpu.make_async_copy(g_hbm.at[ik_v.at[j]],
                                  g_v.at[pl.ds(j * SUB, SUB)], gsem).wait()
        pltpu.make_async_copy(af_src, af_dst, sem).wait()

        def prow(r, carry):
            r8 = r * 8
            for k in range(8):
                g_v[r8 + k, :] = af_v[r8 + k, :] * g_v[r8 + k, :]
            return carry

        lax.fori_loop(0, nrows * SUB // 8, prow, 0)
        for j in range(nrows):
            pltpu.sync_copy(g_v.at[pl.ds(j * SUB, SUB)],
                            acc_sp.at[ij_v.at[j]], add=True)

    for scan in range(2):
        rng = 2 * scan + c
        lo = rng * RSZ
        for q in range(WB // 500):
            pltpu.sync_copy(zero_v, acc_sp.at[pl.ds(s * WB + q * 500, 500)])

        @pl.when(s == 0)
        def _():
            pltpu.sync_copy(zero_v.at[pl.ds(0, 8)], acc_sp.at[pl.ds(RSZ, 8)])

        plsc.subcore_barrier()

        def blk_loop(i, carry):
            b = s + 16 * i

            @pl.when(b < N_BLK)
            def _():
                do_block(b * BROWS, BROWS, lo)

            return carry

        lax.fori_loop(0, (N_BLK + 15) // 16, blk_loop, 0)

        @pl.when(s == 15)
        def _():
            do_block(TAIL_ROW, 1, lo)

        plsc.subcore_barrier()
        for q in range(WB // 500):
            off = s * WB + q * 500
            pltpu.sync_copy(acc_sp.at[pl.ds(off, 500)],
                            acc_hbm.at[pl.ds(lo + off, 500)])
        plsc.subcore_barrier()


def kernel(atomic_features, edge_features, angle_features, edge_dist,
           W_atom, b_atom, W_g, b_g, W_gate, b_gate,
           edge_index, three_body_indices_with_offset):
    filt = _tc_filter(atomic_features, W_atom, b_atom.reshape(1, AD))
    envr = _tc_envrows(edge_dist.reshape(N_EDGES, 1))
    kn = edge_index[1].astype(jnp.int32)
    g = _sc_g(filt, envr, kn)
    tbi = three_body_indices_with_offset.astype(jnp.int32)
    ij = tbi[:, 0].reshape(N_ANGLES // SUB, SUB)
    ik = tbi[:, 1].reshape(N_ANGLES // SUB, SUB)
    acc = _sc_scatter(g, angle_features, ij, ik)
    return _tc_final(acc, envr, edge_features,
                     W_g, b_g.reshape(1, FD), W_gate, b_gate.reshape(1, FD))


# speedup vs baseline: 1.8887x; 1.1156x over previous
import functools

import jax
import jax.numpy as jnp
from jax import lax
from jax.experimental import pallas as pl
from jax.experimental.pallas import tpu as pltpu
from jax.experimental.pallas import tpu_sc as plsc

CUTOFF = 5.0
AD = 16
FD = 128
N_NODES = 10000
N_EDGES = 320000
N_ANGLES = 2000000

SUB = 128
BROWS = 8
BLK = BROWS * SUB
N_BLK = N_ANGLES // BLK
TAIL_ROW = N_ANGLES // SUB - 1
RSZ = 80000
TRASH = RSZ
WB = 5000

_MESH = plsc.VectorSubcoreMesh(core_axis_name="c", subcore_axis_name="s",
                               num_cores=2, num_subcores=16)


def _filter_body(a_ref, w_ref, b_ref, o_ref):
    o_ref[...] = jax.nn.sigmoid(
        jnp.dot(a_ref[...], w_ref[...], preferred_element_type=jnp.float32)
        + b_ref[...])


def _tc_filter(atomic_features, W_atom, b_atom2):
    return pl.pallas_call(
        _filter_body,
        grid=(10,),
        in_specs=[
            pl.BlockSpec((1000, FD), lambda i: (i, 0)),
            pl.BlockSpec((FD, AD), lambda i: (0, 0)),
            pl.BlockSpec((1, AD), lambda i: (0, 0)),
        ],
        out_specs=pl.BlockSpec((1000, AD), lambda i: (i, 0)),
        out_shape=jax.ShapeDtypeStruct((N_NODES, AD), jnp.float32),
    )(atomic_features, W_atom, b_atom2)


def _env_body(d_ref, o_ref):
    r = d_ref[...] * (1.0 / CUTOFF)
    r3 = r * r * r
    env = 1.0 - r3 * ((6.0 * r - 15.0) * r + 10.0)
    o_ref[...] = jnp.broadcast_to(env, (4000, AD))


def _tc_envrows(d_col):
    return pl.pallas_call(
        _env_body,
        grid=(80,),
        in_specs=[pl.BlockSpec((4000, 1), lambda i: (i, 0))],
        out_specs=pl.BlockSpec((4000, AD), lambda i: (i, 0)),
        out_shape=jax.ShapeDtypeStruct((N_EDGES, AD), jnp.float32),
    )(d_col)


def _final_body(acc_ref, env_ref, ef_ref, wg_ref, bg_ref, wt_ref, bt_ref,
                o_ref):
    t = acc_ref[...] * env_ref[...]
    h = jnp.dot(t, wg_ref[...], preferred_element_type=jnp.float32) + bg_ref[...]
    swish = h * jax.nn.sigmoid(h)
    gate = jax.nn.sigmoid(
        jnp.dot(t, wt_ref[...], preferred_element_type=jnp.float32) + bt_ref[...])
    o_ref[...] = ef_ref[...] + swish * gate


def _tc_final(acc, env_col, edge_features, W_g, b_g2, W_gate, b_gate2):
    return pl.pallas_call(
        _final_body,
        grid=(625,),
        in_specs=[
            pl.BlockSpec((512, AD), lambda i: (i, 0)),
            pl.BlockSpec((512, AD), lambda i: (i, 0)),
            pl.BlockSpec((512, FD), lambda i: (i, 0)),
            pl.BlockSpec((AD, FD), lambda i: (0, 0)),
            pl.BlockSpec((1, FD), lambda i: (0, 0)),
            pl.BlockSpec((AD, FD), lambda i: (0, 0)),
            pl.BlockSpec((1, FD), lambda i: (0, 0)),
        ],
        out_specs=pl.BlockSpec((512, FD), lambda i: (i, 0)),
        out_shape=jax.ShapeDtypeStruct((N_EDGES, FD), jnp.float32),
    )(acc, env_col, edge_features, W_g, b_g2, W_gate, b_gate2)


@functools.partial(
    pl.kernel,
    out_type=jax.ShapeDtypeStruct((N_EDGES, AD), jnp.float32),
    mesh=_MESH,
    scratch_types=[
        pltpu.VMEM((SUB,), jnp.int32),
        pltpu.VMEM((SUB, AD), jnp.float32),
        pltpu.VMEM((SUB, AD), jnp.float32),
        pltpu.VMEM((SUB, AD), jnp.float32),
        pltpu.SemaphoreType.DMA,
    ],
    compiler_params=pltpu.CompilerParams(use_tc_tiling_on_sc=False),
)
def _sc_g(filt_hbm, envr_hbm, kn_hbm, g_hbm, kn_v, envr_v, afk_v, g_v, sem):
    c = lax.axis_index("c")
    s = lax.axis_index("s")
    w = c * 16 + s
    n_chunks = N_EDGES // SUB

    def chunk(i, carry):
        cidx = w + 32 * i

        @pl.when(cidx < n_chunks)
        def _():
            base = cidx * SUB
            pltpu.sync_copy(kn_hbm.at[pl.ds(base, SUB)], kn_v)
            pltpu.sync_copy(envr_hbm.at[pl.ds(base, SUB)], envr_v)
            pltpu.async_copy(filt_hbm.at[kn_v], afk_v, sem).wait()

            def row(e, carry2):
                e8 = e * 8
                for k in range(8):
                    g_v[e8 + k, :] = afk_v[e8 + k, :] * envr_v[e8 + k, :]
                return carry2

            lax.fori_loop(0, SUB // 8, row, 0)
            pltpu.sync_copy(g_v, g_hbm.at[pl.ds(base, SUB)])

        return carry

    lax.fori_loop(0, (n_chunks + 31) // 32, chunk, 0)


@functools.partial(
    pl.kernel,
    out_type=jax.ShapeDtypeStruct((N_EDGES, AD), jnp.float32),
    mesh=_MESH,
    scratch_types=[
        pltpu.VMEM((BROWS, SUB), jnp.int32),
        pltpu.VMEM((BROWS, SUB), jnp.int32),
        pltpu.VMEM((BLK, AD), jnp.float32),
        pltpu.VMEM((BLK, AD), jnp.float32),
        pltpu.VMEM((500, AD), jnp.float32),
        pltpu.VMEM_SHARED((RSZ + 8, AD), jnp.float32),
        pltpu.SemaphoreType.DMA,
        pltpu.SemaphoreType.DMA,
    ],
    compiler_params=pltpu.CompilerParams(use_tc_tiling_on_sc=False),
)
def _sc_scatter(g_hbm, af_hbm, ij_hbm, ik_hbm, acc_hbm,
                ij_v, ik_v, af_v, g_v, zero_v, acc_sp, sem, gsem):
    c = lax.axis_index("c")
    s = lax.axis_index("s")
    iota16 = lax.broadcasted_iota(jnp.int32, (16,), 0)
    trash16 = TRASH + (iota16 & 7)
    zv = jnp.zeros((16,), jnp.float32)

    def zrow(r, carry):
        zero_v[r, :] = zv
        return carry

    lax.fori_loop(0, 500, zrow, 0)

    def do_block(row0, nrows, lo):
        pltpu.sync_copy(ij_hbm.at[pl.ds(row0, nrows)], ij_v.at[pl.ds(0, nrows)])
        pltpu.sync_copy(ik_hbm.at[pl.ds(row0, nrows)], ik_v.at[pl.ds(0, nrows)])
        af_src = af_hbm.at[pl.ds(row0 * SUB, nrows * SUB)]
        af_dst = af_v.at[pl.ds(0, nrows * SUB)]
        pltpu.async_copy(af_src, af_dst, sem)
        for j in range(nrows):
            pltpu.async_copy(g_hbm.at[ik_v.at[j]],
                             g_v.at[pl.ds(j * SUB, SUB)], gsem)
        for v in range(nrows * (SUB // 16)):
            j, sl = v // (SUB // 16), (v % (SUB // 16)) * 16
            rel = ij_v[j, pl.ds(sl, 16)] - lo
            msk = (rel >= 0) & (rel < RSZ)
            ij_v[j, pl.ds(sl, 16)] = jnp.where(msk, rel, trash16)
        for j in range(nrows):
            pltpu.make_async_copy(g_hbm.at[ik_v.at[j]],
                                  g_v.at[pl.ds(j * SUB, SUB)], gsem).wait()
        pltpu.make_async_copy(af_src, af_dst, sem).wait()

        def prow(r, carry):
            r8 = r * 8
            for k in range(8):
                g_v[r8 + k, :] = af_v[r8 + k, :] * g_v[r8 + k, :]
            return carry

        lax.fori_loop(0, nrows * SUB // 8, prow, 0)
        for j in range(nrows):
            pltpu.sync_copy(g_v.at[pl.ds(j * SUB, SUB)],
                            acc_sp.at[ij_v.at[j]], add=True)

    for scan in range(2):
        rng = 2 * scan + c
        lo = rng * RSZ
        for q in range(WB // 500):
            pltpu.sync_copy(zero_v, acc_sp.at[pl.ds(s * WB + q * 500, 500)])

        @pl.when(s == 0)
        def _():
            pltpu.sync_copy(zero_v.at[pl.ds(0, 8)], acc_sp.at[pl.ds(RSZ, 8)])

        plsc.subcore_barrier()

        def blk_loop(i, carry):
            b = s + 16 * i

            @pl.when(b < N_BLK)
            def _():
                do_block(b * BROWS, BROWS, lo)

            return carry

        lax.fori_loop(0, (N_BLK + 15) // 16, blk_loop, 0)

        @pl.when(s == 15)
        def _():
            do_block(TAIL_ROW, 1, lo)

        plsc.subcore_barrier()
        for q in range(WB // 500):
            off = s * WB + q * 500
            pltpu.sync_copy(acc_sp.at[pl.ds(off, 500)],
                            acc_hbm.at[pl.ds(lo + off, 500)])
        plsc.subcore_barrier()


def kernel(atomic_features, edge_features, angle_features, edge_dist,
           W_atom, b_atom, W_g, b_g, W_gate, b_gate,
           edge_index, three_body_indices_with_offset):
    filt = _tc_filter(atomic_features, W_atom, b_atom.reshape(1, AD))
    envr = _tc_envrows(edge_dist.reshape(N_EDGES, 1))
    kn = edge_index[1].astype(jnp.int32)
    g = _sc_g(filt, envr, kn)
    tbi = three_body_indices_with_offset.astype(jnp.int32)
    ij = tbi[:, 0].reshape(N_ANGLES // SUB, SUB)
    ik = tbi[:, 1].reshape(N_ANGLES // SUB, SUB)
    acc = _sc_scatter(g, angle_features, ij, ik)
    return _tc_final(acc, envr, edge_features,
                     W_g, b_g.reshape(1, FD), W_gate, b_gate.reshape(1, FD))


# speedup vs baseline: 1.9570x; 1.0361x over previous
import functools

import jax
import jax.numpy as jnp
from jax import lax
from jax.experimental import pallas as pl
from jax.experimental.pallas import tpu as pltpu
from jax.experimental.pallas import tpu_sc as plsc

CUTOFF = 5.0
AD = 16
FD = 128
N_NODES = 10000
N_EDGES = 320000
N_ANGLES = 2000000

SUB = 128
BROWS = 8
BLK = BROWS * SUB
N_BLK = N_ANGLES // BLK
TAIL_ROW = N_ANGLES // SUB - 1
RSZ = 80000
TRASH = RSZ
WB = 5000

_MESH = plsc.VectorSubcoreMesh(core_axis_name="c", subcore_axis_name="s",
                               num_cores=2, num_subcores=16)


def _filter_body(a_ref, w_ref, b_ref, o_ref):
    o_ref[...] = jax.nn.sigmoid(
        jnp.dot(a_ref[...], w_ref[...], preferred_element_type=jnp.float32)
        + b_ref[...])


def _tc_filter(atomic_features, W_atom, b_atom2):
    return pl.pallas_call(
        _filter_body,
        grid=(10,),
        in_specs=[
            pl.BlockSpec((1000, FD), lambda i: (i, 0)),
            pl.BlockSpec((FD, AD), lambda i: (0, 0)),
            pl.BlockSpec((1, AD), lambda i: (0, 0)),
        ],
        out_specs=pl.BlockSpec((1000, AD), lambda i: (i, 0)),
        out_shape=jax.ShapeDtypeStruct((N_NODES, AD), jnp.float32),
    )(atomic_features, W_atom, b_atom2)


def _env_body(d_ref, o_ref):
    r = d_ref[...] * (1.0 / CUTOFF)
    r3 = r * r * r
    env = 1.0 - r3 * ((6.0 * r - 15.0) * r + 10.0)
    o_ref[...] = jnp.broadcast_to(env, (4000, AD))


def _tc_envrows(d_col):
    return pl.pallas_call(
        _env_body,
        grid=(80,),
        in_specs=[pl.BlockSpec((4000, 1), lambda i: (i, 0))],
        out_specs=pl.BlockSpec((4000, AD), lambda i: (i, 0)),
        out_shape=jax.ShapeDtypeStruct((N_EDGES, AD), jnp.float32),
    )(d_col)


def _final_body(acc_ref, env_ref, ef_ref, wg_ref, bg_ref, wt_ref, bt_ref,
                o_ref):
    t = acc_ref[...] * env_ref[...]
    h = jnp.dot(t, wg_ref[...], preferred_element_type=jnp.float32) + bg_ref[...]
    swish = h * jax.nn.sigmoid(h)
    gate = jax.nn.sigmoid(
        jnp.dot(t, wt_ref[...], preferred_element_type=jnp.float32) + bt_ref[...])
    o_ref[...] = ef_ref[...] + swish * gate


def _tc_final(acc, env_col, edge_features, W_g, b_g2, W_gate, b_gate2):
    return pl.pallas_call(
        _final_body,
        grid=(625,),
        in_specs=[
            pl.BlockSpec((512, AD), lambda i: (i, 0)),
            pl.BlockSpec((512, AD), lambda i: (i, 0)),
            pl.BlockSpec((512, FD), lambda i: (i, 0)),
            pl.BlockSpec((AD, FD), lambda i: (0, 0)),
            pl.BlockSpec((1, FD), lambda i: (0, 0)),
            pl.BlockSpec((AD, FD), lambda i: (0, 0)),
            pl.BlockSpec((1, FD), lambda i: (0, 0)),
        ],
        out_specs=pl.BlockSpec((512, FD), lambda i: (i, 0)),
        out_shape=jax.ShapeDtypeStruct((N_EDGES, FD), jnp.float32),
    )(acc, env_col, edge_features, W_g, b_g2, W_gate, b_gate2)


@functools.partial(
    pl.kernel,
    out_type=jax.ShapeDtypeStruct((N_EDGES, AD), jnp.float32),
    mesh=_MESH,
    scratch_types=[
        pltpu.VMEM((SUB,), jnp.int32),
        pltpu.VMEM((SUB, AD), jnp.float32),
        pltpu.VMEM((SUB, AD), jnp.float32),
        pltpu.VMEM((SUB, AD), jnp.float32),
        pltpu.SemaphoreType.DMA,
    ],
    compiler_params=pltpu.CompilerParams(use_tc_tiling_on_sc=False),
)
def _sc_g(filt_hbm, envr_hbm, kn_hbm, g_hbm, kn_v, envr_v, afk_v, g_v, sem):
    c = lax.axis_index("c")
    s = lax.axis_index("s")
    w = c * 16 + s
    n_chunks = N_EDGES // SUB

    def chunk(i, carry):
        cidx = w + 32 * i

        @pl.when(cidx < n_chunks)
        def _():
            base = cidx * SUB
            pltpu.sync_copy(kn_hbm.at[pl.ds(base, SUB)], kn_v)
            pltpu.sync_copy(envr_hbm.at[pl.ds(base, SUB)], envr_v)
            pltpu.async_copy(filt_hbm.at[kn_v], afk_v, sem).wait()

            def row(e, carry2):
                e8 = e * 8
                for k in range(8):
                    g_v[e8 + k, :] = afk_v[e8 + k, :] * envr_v[e8 + k, :]
                return carry2

            lax.fori_loop(0, SUB // 8, row, 0)
            pltpu.sync_copy(g_v, g_hbm.at[pl.ds(base, SUB)])

        return carry

    lax.fori_loop(0, (n_chunks + 31) // 32, chunk, 0)


@functools.partial(
    pl.kernel,
    out_type=jax.ShapeDtypeStruct((N_EDGES, AD), jnp.float32),
    mesh=_MESH,
    scratch_types=[
        pltpu.VMEM((BROWS, SUB), jnp.int32),
        pltpu.VMEM((BROWS, SUB), jnp.int32),
        pltpu.VMEM((BLK, AD), jnp.float32),
        pltpu.VMEM((BLK, AD), jnp.float32),
        pltpu.VMEM((500, AD), jnp.float32),
        pltpu.VMEM_SHARED((RSZ + 8, AD), jnp.float32),
        pltpu.SemaphoreType.DMA,
        pltpu.SemaphoreType.DMA,
    ],
    compiler_params=pltpu.CompilerParams(use_tc_tiling_on_sc=False),
)
def _sc_scatter(g_hbm, af_hbm, ij_hbm, ik_hbm, acc_hbm,
                ij_v, ik_v, af_v, g_v, zero_v, acc_sp, sem, gsem):
    c = lax.axis_index("c")
    s = lax.axis_index("s")
    iota16 = lax.broadcasted_iota(jnp.int32, (16,), 0)
    trash16 = TRASH + (iota16 & 7)
    zv = jnp.zeros((16,), jnp.float32)

    def zrow(r, carry):
        zero_v[r, :] = zv
        return carry

    lax.fori_loop(0, 500, zrow, 0)

    def do_block(row0, nrows, lo):
        pltpu.sync_copy(ik_hbm.at[pl.ds(row0, nrows)], ik_v.at[pl.ds(0, nrows)])
        af_src = af_hbm.at[pl.ds(row0 * SUB, nrows * SUB)]
        af_dst = af_v.at[pl.ds(0, nrows * SUB)]
        pltpu.async_copy(af_src, af_dst, sem)
        for j in range(nrows):
            pltpu.async_copy(g_hbm.at[ik_v.at[j]],
                             g_v.at[pl.ds(j * SUB, SUB)], gsem)
        pltpu.sync_copy(ij_hbm.at[pl.ds(row0, nrows)], ij_v.at[pl.ds(0, nrows)])
        for v in range(nrows * (SUB // 16)):
            j, sl = v // (SUB // 16), (v % (SUB // 16)) * 16
            rel = ij_v[j, pl.ds(sl, 16)] - lo
            msk = (rel >= 0) & (rel < RSZ)
            ij_v[j, pl.ds(sl, 16)] = jnp.where(msk, rel, trash16)
        for j in range(nrows):
            pltpu.make_async_copy(g_hbm.at[ik_v.at[j]],
                                  g_v.at[pl.ds(j * SUB, SUB)], gsem).wait()
        pltpu.make_async_copy(af_src, af_dst, sem).wait()

        def prow(r, carry):
            r8 = r * 8
            for k in range(8):
                g_v[r8 + k, :] = af_v[r8 + k, :] * g_v[r8 + k, :]
            return carry

        lax.fori_loop(0, nrows * SUB // 8, prow, 0)
        for j in range(nrows):
            pltpu.sync_copy(g_v.at[pl.ds(j * SUB, SUB)],
                            acc_sp.at[ij_v.at[j]], add=True)

    for scan in range(2):
        rng = 2 * scan + c
        lo = rng * RSZ
        for q in range(WB // 500):
            pltpu.sync_copy(zero_v, acc_sp.at[pl.ds(s * WB + q * 500, 500)])

        @pl.when(s == 0)
        def _():
            pltpu.sync_copy(zero_v.at[pl.ds(0, 8)], acc_sp.at[pl.ds(RSZ, 8)])

        plsc.subcore_barrier()

        def blk_loop(i, carry):
            b = s + 16 * i

            @pl.when(b < N_BLK)
            def _():
                do_block(b * BROWS, BROWS, lo)

            return carry

        lax.fori_loop(0, (N_BLK + 15) // 16, blk_loop, 0)

        @pl.when(s == 15)
        def _():
            do_block(TAIL_ROW, 1, lo)

        plsc.subcore_barrier()
        for q in range(WB // 500):
            off = s * WB + q * 500
            pltpu.sync_copy(acc_sp.at[pl.ds(off, 500)],
                            acc_hbm.at[pl.ds(lo + off, 500)])
        plsc.subcore_barrier()


def kernel(atomic_features, edge_features, angle_features, edge_dist,
           W_atom, b_atom, W_g, b_g, W_gate, b_gate,
           edge_index, three_body_indices_with_offset):
    filt = _tc_filter(atomic_features, W_atom, b_atom.reshape(1, AD))
    envr = _tc_envrows(edge_dist.reshape(N_EDGES, 1))
    kn = edge_index[1].astype(jnp.int32)
    g = _sc_g(filt, envr, kn)
    tbi = three_body_indices_with_offset.astype(jnp.int32)
    ij = tbi[:, 0].reshape(N_ANGLES // SUB, SUB)
    ik = tbi[:, 1].reshape(N_ANGLES // SUB, SUB)
    acc = _sc_scatter(g, angle_features, ij, ik)
    return _tc_final(acc, envr, edge_features,
                     W_g, b_g.reshape(1, FD), W_gate, b_gate.reshape(1, FD))
